# Initial kernel scaffold; baseline (speedup 1.0000x reference)
import jax
import jax.numpy as jnp
from jax.experimental import pallas as pl


def kernel(x, embedding):
    raise NotImplementedError("write your pallas kernel here")


# speedup vs baseline: 1.6933x; 1.6933x over previous
import jax
import jax.numpy as jnp
from jax.experimental import pallas as pl


def _add_kernel(x_ref, emb_ref, out_ref):
    out_ref[...] = x_ref[...] + emb_ref[...]


def kernel(x, embedding):
    b, c, d = x.shape
    bc = 512
    nc = c // bc

    grid = (nc, b)
    return pl.pallas_call(
        _add_kernel,
        grid=grid,
        in_specs=[
            pl.BlockSpec((1, bc, d), lambda ci, bi: (bi, ci, 0)),
            pl.BlockSpec((bc, d), lambda ci, bi: (ci, 0)),
        ],
        out_specs=pl.BlockSpec((1, bc, d), lambda ci, bi: (bi, ci, 0)),
        out_shape=jax.ShapeDtypeStruct((b, c, d), x.dtype),
    )(x, embedding)


# speedup vs baseline: 1.9656x; 1.1608x over previous
import jax
import jax.numpy as jnp
from jax.experimental import pallas as pl


def _add_kernel(x_ref, emb_ref, out_ref):
    out_ref[...] = x_ref[...] + emb_ref[...][None]


def kernel(x, embedding):
    b, c, d = x.shape
    bc = 512
    nc = c // bc

    grid = (nc,)
    return pl.pallas_call(
        _add_kernel,
        grid=grid,
        in_specs=[
            pl.BlockSpec((b, bc, d), lambda ci: (0, ci, 0)),
            pl.BlockSpec((bc, d), lambda ci: (ci, 0)),
        ],
        out_specs=pl.BlockSpec((b, bc, d), lambda ci: (0, ci, 0)),
        out_shape=jax.ShapeDtypeStruct((b, c, d), x.dtype),
    )(x, embedding)
